# baseline (speedup 1.0000x reference)
import functools
import numpy as np
import jax
import jax.numpy as jnp
from jax import lax
from jax.experimental import pallas as pl
from jax.experimental.pallas import tpu as pltpu
from jax.experimental.pallas import tpu_sc as plsc

_VOCAB = 100000
_D = 1024
_SEQ = 2048
_B = 4

_NC = 2
_NS = 16
_NW = _NC * _NS
_ROWS = _SEQ * _B
_RPW = _ROWS // _NW
_CH = 64
_NCH = _RPW // _CH
_SPC = _CH // _B
_LANES = 16


def _sinusoidal_pe(max_len, d):
    position = np.arange(max_len, dtype=np.float32)[:, None]
    div_term = np.exp(np.arange(0, d, 2).astype(np.float32) * -(np.log(10000.0) / d))
    enc = np.zeros((max_len, d), dtype=np.float32)
    enc[:, 0::2] = np.sin(position * div_term)
    enc[:, 1::2] = np.cos(position * div_term)
    return enc


_PE_NP = _sinusoidal_pe(_SEQ, _D)


def _body(table_hbm, idx_hbm, pe_hbm, out_hbm, idx_v, pe_v, rows_v, sem):
    wid = lax.axis_index("s") * _NC + lax.axis_index("c")
    pltpu.sync_copy(idx_hbm.at[pl.ds(wid * _NCH, _NCH)], idx_v)

    def chunk(c, carry):
        pltpu.async_copy(table_hbm.at[idx_v.at[c]], rows_v, sem).wait()
        seq0 = wid * (_RPW // _B) + c * _SPC
        pltpu.sync_copy(pe_hbm.at[pl.ds(seq0, _SPC)], pe_v)

        def seq_add(s, carry2):
            def vec_add(v, carry3):
                pe_vec = pe_v[s, pl.ds(v * _LANES, _LANES)]
                for b in range(_B):
                    r = s * _B + b
                    rows_v[r, pl.ds(v * _LANES, _LANES)] += pe_vec
                return carry3
            return lax.fori_loop(0, _D // _LANES, vec_add, carry2)

        lax.fori_loop(0, _SPC, seq_add, None)

        pltpu.sync_copy(rows_v, out_hbm.at[pl.ds(wid * _RPW + c * _CH, _CH)])
        return carry

    lax.fori_loop(0, _NCH, chunk, None)


@jax.jit
def kernel(x, token_table):
    idx2d = x.reshape(_NW * _NCH, _CH)
    mesh = plsc.VectorSubcoreMesh(core_axis_name="c", subcore_axis_name="s")
    k = pl.kernel(
        _body,
        mesh=mesh,
        out_type=jax.ShapeDtypeStruct((_ROWS, _D), jnp.float32),
        scratch_types=[
            pltpu.VMEM((_NW * _NCH // _NW, _CH), jnp.int32),
            pltpu.VMEM((_SPC, _D), jnp.float32),
            pltpu.VMEM((_CH, _D), jnp.float32),
            pltpu.SemaphoreType.DMA,
        ],
    )
    out = k(token_table, idx2d, jnp.asarray(_PE_NP))
    return out.reshape(_SEQ, _B, _D)


# speedup vs baseline: 1.2390x; 1.2390x over previous
import functools
import numpy as np
import jax
import jax.numpy as jnp
from jax import lax
from jax.experimental import pallas as pl
from jax.experimental.pallas import tpu as pltpu
from jax.experimental.pallas import tpu_sc as plsc

_VOCAB = 100000
_D = 1024
_SEQ = 2048
_B = 4

_NC = 2
_NS = 16
_NW = _NC * _NS
_ROWS = _SEQ * _B
_RPW = _ROWS // _NW
_CH = 32
_NCH = _RPW // _CH
_SPC = _CH // _B
_LANES = 16


def _sinusoidal_pe(max_len, d):
    position = np.arange(max_len, dtype=np.float32)[:, None]
    div_term = np.exp(np.arange(0, d, 2).astype(np.float32) * -(np.log(10000.0) / d))
    enc = np.zeros((max_len, d), dtype=np.float32)
    enc[:, 0::2] = np.sin(position * div_term)
    enc[:, 1::2] = np.cos(position * div_term)
    return enc


_PE_NP = _sinusoidal_pe(_SEQ, _D)


def _body(table_hbm, idx_hbm, pe_hbm, out_hbm,
          idx_v, rows0, rows1, pe0, pe1, gsem0, gsem1, psem0, psem1,
          osem0, osem1):
    wid = lax.axis_index("s") * _NC + lax.axis_index("c")
    rows = (rows0, rows1)
    pes = (pe0, pe1)
    gsems = (gsem0, gsem1)
    psems = (psem0, psem1)
    osems = (osem0, osem1)

    pltpu.sync_copy(idx_hbm.at[pl.ds(wid * _NCH, _NCH)], idx_v)

    def start_fetch(g, b):
        pltpu.async_copy(table_hbm.at[idx_v.at[g]], rows[b], gsems[b])
        seq0 = wid * (_RPW // _B) + g * _SPC
        pltpu.async_copy(pe_hbm.at[pl.ds(seq0, _SPC)], pes[b], psems[b])

    def wait_fetch(b):
        pltpu.make_async_copy(table_hbm.at[idx_v.at[0]], rows[b], gsems[b]).wait()
        pltpu.make_async_copy(pe_hbm.at[pl.ds(0, _SPC)], pes[b], psems[b]).wait()

    def wait_store(b):
        pltpu.make_async_copy(rows[b], out_hbm.at[pl.ds(0, _CH)], osems[b]).wait()

    start_fetch(0, 0)

    def outer(c0, carry):
        for b in range(2):
            g = c0 * 2 + b
            nb = 1 - b
            wait_fetch(b)

            @pl.when(g + 1 < _NCH)
            def _prefetch():
                @pl.when(g >= 1)
                def _drain():
                    wait_store(nb)
                start_fetch(g + 1, nb)

            rows_b = rows[b]
            pe_b = pes[b]

            def seq_add(s, carry2):
                def vec_add(v4, carry3):
                    for vv in range(4):
                        col = (v4 * 4 + vv) * _LANES
                        pe_vec = pe_b[s, pl.ds(col, _LANES)]
                        for bb in range(_B):
                            plsc.addupdate(
                                rows_b.at[s * _B + bb, pl.ds(col, _LANES)],
                                pe_vec)
                    return carry3
                return lax.fori_loop(0, _D // (4 * _LANES), vec_add, carry2)

            lax.fori_loop(0, _SPC, seq_add, None)

            pltpu.async_copy(
                rows_b, out_hbm.at[pl.ds(wid * _RPW + g * _CH, _CH)], osems[b])
        return carry

    lax.fori_loop(0, _NCH // 2, outer, None)
    wait_store(0)
    wait_store(1)


@jax.jit
def kernel(x, token_table):
    idx2d = x.reshape(_NW * _NCH, _CH)
    mesh = plsc.VectorSubcoreMesh(core_axis_name="c", subcore_axis_name="s")
    k = pl.kernel(
        _body,
        mesh=mesh,
        out_type=jax.ShapeDtypeStruct((_ROWS, _D), jnp.float32),
        scratch_types=[
            pltpu.VMEM((_NCH, _CH), jnp.int32),
            pltpu.VMEM((_CH, _D), jnp.float32),
            pltpu.VMEM((_CH, _D), jnp.float32),
            pltpu.VMEM((_SPC, _D), jnp.float32),
            pltpu.VMEM((_SPC, _D), jnp.float32),
            pltpu.SemaphoreType.DMA,
            pltpu.SemaphoreType.DMA,
            pltpu.SemaphoreType.DMA,
            pltpu.SemaphoreType.DMA,
            pltpu.SemaphoreType.DMA,
            pltpu.SemaphoreType.DMA,
        ],
    )
    out = k(token_table, idx2d, jnp.asarray(_PE_NP))
    return out.reshape(_SEQ, _B, _D)
